# baseline (speedup 1.0000x reference)
import functools
import math

import jax
import jax.numpy as jnp
from jax import lax
from jax.experimental import pallas as pl
from jax.experimental.pallas import tpu as pltpu
from jax.experimental.pallas import tpu_sc as plsc

EMBED = 64
SCALE = math.sqrt(EMBED)

NC = 2
NS = 16
NW = NC * NS

CHUNK = 128
NBUF = 5
LANES = 16
VPR = CHUNK // LANES
EVG = EMBED // LANES
SPAD = 129


def _body(npos, tokt_hbm, table_hbm, out_hbm, idx_v,
          i0, i1, i2, i3, i4, o0, o1, o2, o3, o4,
          g0, g1, g2, g3, g4, s0, s1, s2, s3, s4):
  ib = (i0, i1, i2, i3, i4)
  ob = (o0, o1, o2, o3, o4)
  gs = (g0, g1, g2, g3, g4)
  os_ = (s0, s1, s2, s3, s4)

  wid = lax.axis_index("s") * NC + lax.axis_index("c")

  pltpu.sync_copy(tokt_hbm.at[:, pl.ds(wid * CHUNK, CHUNK)], idx_v)

  iota = lax.iota(jnp.int32, LANES)
  DBV = [(iota + 16 * k) >> 3 for k in range(EVG)]
  DRV = [(iota + 16 * k) & 7 for k in range(EVG)]

  for b in range(NBUF):
    pltpu.async_copy(table_hbm.at[idx_v.at[b]], ib[b], gs[b])

  @pl.loop(0, npos, step=NBUF)
  def _(p0):
    for b in range(NBUF):
      p = p0 + b
      pltpu.make_async_copy(table_hbm.at[idx_v.at[p]], ib[b], gs[b]).wait()

      @pl.when(p0 > 0)
      def _():
        pltpu.make_async_copy(
            ob[b].at[:, :, pl.ds(0, CHUNK)],
            out_hbm.at[p, pl.ds(0, 8), wid], os_[b]).wait()

      @pl.loop(0, CHUNK, step=16)
      def _(s0):
        seq = [(ds_, k) for ds_ in range(16) for k in range(EVG)]
        nsq = len(seq)
        P = 8
        lds = {}
        cvs = {}

        def _ld(i):
          ds_, k = seq[i]
          if k == 0:
            cvs[ds_] = jnp.full((LANES,), s0 + ds_, jnp.int32)
          lds[i] = ib[b][s0 + ds_, pl.ds(k * LANES, LANES)]

        def _st(i):
          ds_, k = seq[i]
          plsc.store_scatter(ob[b], [DBV[k], DRV[k], cvs[ds_]], lds.pop(i))

        for i in range(nsq):
          _ld(i)
          if i >= P:
            _st(i - P)
        for i in range(nsq - P, nsq):
          _st(i)

      @pl.when(p0 + 2 * NBUF <= npos)
      def _():
        pltpu.async_copy(table_hbm.at[idx_v.at[p + NBUF]], ib[b], gs[b])

      pltpu.async_copy(ob[b].at[:, :, pl.ds(0, CHUNK)],
                       out_hbm.at[p, pl.ds(0, 8), wid], os_[b])

  for b in range(NBUF):
    pltpu.make_async_copy(ob[b].at[:, :, pl.ds(0, CHUNK)],
                          out_hbm.at[0, pl.ds(0, 8), wid], os_[b]).wait()


PW = 16000
PL = 8000
NPB = 63


def _prep_body(tt, ttail, outp, v0, v1, vt, w0, w1, si0, si1, so0, so1):
  i = pl.program_id(0)
  vin = (v0, v1)
  vout = (w0, w1)
  si = (si0, si1)
  so = (so0, so1)

  def in_cp(j, s):
    return pltpu.make_async_copy(
        tt.at[:, pl.ds(j * PW, PW)], vin[s], si[s])

  def tail_cp(s):
    return pltpu.make_async_copy(ttail, vt, si[s])

  def out_cp(j, s, w):
    return pltpu.make_async_copy(
        vout[s].at[pl.ds(0, w), :], outp.at[pl.ds(j * PW, w), :], so[s])

  @pl.when(i == 0)
  def _():
    in_cp(0, 0).start()
    in_cp(1, 1).start()

  def run(s):
    @pl.when(i < NPB - 1)
    def _():
      in_cp(i, s).wait()

    @pl.when(i == NPB - 1)
    def _():
      tail_cp(s).wait()

    @pl.when(i >= 2)
    def _():
      out_cp(i - 2, s, PW).wait()

    @pl.when(i < NPB - 1)
    def _():
      vout[s][:, :EMBED] = jnp.swapaxes(vin[s][...], 0, 1) * SCALE
      out_cp(i, s, PW).start()

      @pl.when(i + 2 < NPB - 1)
      def _():
        in_cp(i + 2, s).start()

      @pl.when(i + 2 == NPB - 1)
      def _():
        tail_cp(s).start()

    @pl.when(i == NPB - 1)
    def _():
      vout[s][pl.ds(0, PL), :EMBED] = jnp.swapaxes(vt[...], 0, 1) * SCALE
      out_cp(i, s, PL).start()
      out_cp(i - 1, 1 - s, PW).wait()
      out_cp(i, s, PL).wait()

  @pl.when(i % 2 == 0)
  def _():
    run(0)

  @pl.when(i % 2 == 1)
  def _():
    run(1)


@jax.jit
def _prep_table(tt, ttail):
  nvoc = tt.shape[1]
  return pl.pallas_call(
      _prep_body,
      grid=(NPB,),
      in_specs=[pl.BlockSpec(memory_space=pl.ANY),
                pl.BlockSpec(memory_space=pl.ANY)],
      out_specs=pl.BlockSpec(memory_space=pl.ANY),
      out_shape=jax.ShapeDtypeStruct((nvoc, 2 * EMBED), jnp.float32),
      scratch_shapes=[
          pltpu.VMEM((EMBED, PW), jnp.float32),
          pltpu.VMEM((EMBED, PW), jnp.float32),
          pltpu.VMEM((EMBED, PL), jnp.float32),
          pltpu.VMEM((PW, 2 * EMBED), jnp.float32),
          pltpu.VMEM((PW, 2 * EMBED), jnp.float32),
          pltpu.SemaphoreType.DMA,
          pltpu.SemaphoreType.DMA,
          pltpu.SemaphoreType.DMA,
          pltpu.SemaphoreType.DMA,
      ],
  )(tt, ttail)


@functools.partial(jax.jit, static_argnames=("npos",))
def _embed_sc(tokt, table, npos):
  mesh = plsc.VectorSubcoreMesh(core_axis_name="c", subcore_axis_name="s")
  f = pl.kernel(
      functools.partial(_body, npos),
      out_type=jax.ShapeDtypeStruct((npos, 8, NW, 8, CHUNK), jnp.float32),
      mesh=mesh,
      compiler_params=pltpu.CompilerParams(
          use_tc_tiling_on_sc=False, needs_layout_passes=False),
      scratch_types=(
          [pltpu.VMEM((npos, CHUNK), jnp.int32)]
          + [pltpu.VMEM((CHUNK, EMBED), jnp.float32)] * NBUF
          + [pltpu.VMEM((8, 8, SPAD), jnp.float32)] * NBUF
          + [pltpu.SemaphoreType.DMA] * (2 * NBUF)
      ),
  )
  return f(tokt, table)


def kernel(tokens, table):
  nseq, npos = tokens.shape
  nvoc = table.shape[0]
  tt = table.T
  tpad = _prep_table(tt, tt[:, nvoc - PL:])
  tbl2 = tpad.reshape(2 * nvoc, EMBED)
  tokt = (tokens.T.astype(jnp.int32) * 2)
  out5 = _embed_sc(tokt, tbl2, npos)
  t = out5.transpose((2, 4, 0, 1, 3))
  return t.reshape(nseq, npos, EMBED)


# speedup vs baseline: 1.0115x; 1.0115x over previous
import functools
import math

import jax
import jax.numpy as jnp
from jax import lax
from jax.experimental import pallas as pl
from jax.experimental.pallas import tpu as pltpu
from jax.experimental.pallas import tpu_sc as plsc

EMBED = 64
SCALE = math.sqrt(EMBED)

NC = 2
NS = 16
NW = NC * NS

CHUNK = 128
NBUF = 5
LANES = 16
VPR = CHUNK // LANES
EVG = EMBED // LANES
SPAD = 129


def _body(npos, tokt_hbm, table_hbm, out_hbm, idx_v,
          i0, i1, i2, i3, i4, o0, o1, o2, o3, o4,
          g0, g1, g2, g3, g4, s0, s1, s2, s3, s4):
  ib = (i0, i1, i2, i3, i4)
  ob = (o0, o1, o2, o3, o4)
  gs = (g0, g1, g2, g3, g4)
  os_ = (s0, s1, s2, s3, s4)

  wid = lax.axis_index("s") * NC + lax.axis_index("c")

  pltpu.sync_copy(tokt_hbm.at[:, pl.ds(wid * CHUNK, CHUNK)], idx_v)

  iota = lax.iota(jnp.int32, LANES)
  DBV = [(iota + 16 * k) >> 3 for k in range(EVG)]
  DRV = [(iota + 16 * k) & 7 for k in range(EVG)]

  for b in range(NBUF):
    pltpu.async_copy(table_hbm.at[idx_v.at[b]], ib[b], gs[b])

  @pl.loop(0, npos, step=NBUF)
  def _(p0):
    for b in range(NBUF):
      p = p0 + b
      pltpu.make_async_copy(table_hbm.at[idx_v.at[p]], ib[b], gs[b]).wait()

      @pl.when(p0 > 0)
      def _():
        pltpu.make_async_copy(
            ob[b].at[:, :, pl.ds(0, CHUNK)],
            out_hbm.at[p, pl.ds(0, 8), wid], os_[b]).wait()

      @pl.loop(0, CHUNK, step=16)
      def _(s0):
        seq = [(ds_, k) for ds_ in range(16) for k in range(EVG)]
        nsq = len(seq)
        P = 8
        lds = {}
        cvs = {}

        def _ld(i):
          ds_, k = seq[i]
          if k == 0:
            cvs[ds_] = jnp.full((LANES,), s0 + ds_, jnp.int32)
          lds[i] = ib[b][s0 + ds_, pl.ds(k * LANES, LANES)]

        def _st(i):
          ds_, k = seq[i]
          plsc.store_scatter(ob[b], [DBV[k], DRV[k], cvs[ds_]], lds.pop(i))

        for i in range(nsq):
          _ld(i)
          if i >= P:
            _st(i - P)
        for i in range(nsq - P, nsq):
          _st(i)

      @pl.when(p0 + 2 * NBUF <= npos)
      def _():
        pltpu.async_copy(table_hbm.at[idx_v.at[p + NBUF]], ib[b], gs[b])

      pltpu.async_copy(ob[b].at[:, :, pl.ds(0, CHUNK)],
                       out_hbm.at[p, pl.ds(0, 8), wid], os_[b])

  for b in range(NBUF):
    pltpu.make_async_copy(ob[b].at[:, :, pl.ds(0, CHUNK)],
                          out_hbm.at[0, pl.ds(0, 8), wid], os_[b]).wait()


PW = 32000
PL = 8000
NPB = 32


def _prep_body(tt, ttail, outp, v0, v1, vt, w0, w1, si0, si1, so0, so1):
  i = pl.program_id(0)
  vin = (v0, v1)
  vout = (w0, w1)
  si = (si0, si1)
  so = (so0, so1)

  def in_cp(j, s):
    return pltpu.make_async_copy(
        tt.at[:, pl.ds(j * PW, PW)], vin[s], si[s])

  def tail_cp(s):
    return pltpu.make_async_copy(ttail, vt, si[s])

  def out_cp(j, s, w):
    return pltpu.make_async_copy(
        vout[s].at[pl.ds(0, w), :], outp.at[pl.ds(j * PW, w), :], so[s])

  @pl.when(i == 0)
  def _():
    in_cp(0, 0).start()
    in_cp(1, 1).start()

  def run(s):
    @pl.when(i < NPB - 1)
    def _():
      in_cp(i, s).wait()

    @pl.when(i == NPB - 1)
    def _():
      tail_cp(s).wait()

    @pl.when(i >= 2)
    def _():
      out_cp(i - 2, s, PW).wait()

    @pl.when(i < NPB - 1)
    def _():
      vout[s][:, :EMBED] = jnp.swapaxes(vin[s][...], 0, 1) * SCALE
      out_cp(i, s, PW).start()

      @pl.when(i + 2 < NPB - 1)
      def _():
        in_cp(i + 2, s).start()

      @pl.when(i + 2 == NPB - 1)
      def _():
        tail_cp(s).start()

    @pl.when(i == NPB - 1)
    def _():
      vout[s][pl.ds(0, PL), :EMBED] = jnp.swapaxes(vt[...], 0, 1) * SCALE
      out_cp(i, s, PL).start()
      out_cp(i - 1, 1 - s, PW).wait()
      out_cp(i, s, PL).wait()

  @pl.when(i % 2 == 0)
  def _():
    run(0)

  @pl.when(i % 2 == 1)
  def _():
    run(1)


@jax.jit
def _prep_table(tt, ttail):
  nvoc = tt.shape[1]
  return pl.pallas_call(
      _prep_body,
      grid=(NPB,),
      in_specs=[pl.BlockSpec(memory_space=pl.ANY),
                pl.BlockSpec(memory_space=pl.ANY)],
      out_specs=pl.BlockSpec(memory_space=pl.ANY),
      out_shape=jax.ShapeDtypeStruct((nvoc, 2 * EMBED), jnp.float32),
      scratch_shapes=[
          pltpu.VMEM((EMBED, PW), jnp.float32),
          pltpu.VMEM((EMBED, PW), jnp.float32),
          pltpu.VMEM((EMBED, PL), jnp.float32),
          pltpu.VMEM((PW, 2 * EMBED), jnp.float32),
          pltpu.VMEM((PW, 2 * EMBED), jnp.float32),
          pltpu.SemaphoreType.DMA,
          pltpu.SemaphoreType.DMA,
          pltpu.SemaphoreType.DMA,
          pltpu.SemaphoreType.DMA,
      ],
  )(tt, ttail)


@functools.partial(jax.jit, static_argnames=("npos",))
def _embed_sc(tokt, table, npos):
  mesh = plsc.VectorSubcoreMesh(core_axis_name="c", subcore_axis_name="s")
  f = pl.kernel(
      functools.partial(_body, npos),
      out_type=jax.ShapeDtypeStruct((npos, 8, NW, 8, CHUNK), jnp.float32),
      mesh=mesh,
      compiler_params=pltpu.CompilerParams(
          use_tc_tiling_on_sc=False, needs_layout_passes=False),
      scratch_types=(
          [pltpu.VMEM((npos, CHUNK), jnp.int32)]
          + [pltpu.VMEM((CHUNK, EMBED), jnp.float32)] * NBUF
          + [pltpu.VMEM((8, 8, SPAD), jnp.float32)] * NBUF
          + [pltpu.SemaphoreType.DMA] * (2 * NBUF)
      ),
  )
  return f(tokt, table)


def kernel(tokens, table):
  nseq, npos = tokens.shape
  nvoc = table.shape[0]
  tt = table.T
  tpad = _prep_table(tt, tt[:, nvoc - PL:])
  tbl2 = tpad.reshape(2 * nvoc, EMBED)
  tokt = (tokens.T.astype(jnp.int32) * 2)
  out5 = _embed_sc(tokt, tbl2, npos)
  t = out5.transpose((2, 4, 0, 1, 3))
  return t.reshape(nseq, npos, EMBED)
